# baseline (speedup 1.0000x reference)
import functools

import jax
import jax.numpy as jnp
from jax.experimental import pallas as pl
from jax.experimental.pallas import tpu as pltpu

_IGNORE = -100


def _focal_body(logits_ref, tgt_ref, out_ref, loss_ref, cnt_ref, *, n_steps):
    i = pl.program_id(1)

    @pl.when(i == 0)
    def _init():
        loss_ref[...] = jnp.zeros_like(loss_ref)
        cnt_ref[...] = jnp.zeros_like(cnt_ref)

    x = logits_ref[...]
    tgt = tgt_ref[...]

    m = jnp.max(x, axis=-1, keepdims=True)
    e = jnp.exp(x - m)
    se = jnp.sum(e, axis=-1, keepdims=True)
    col = jax.lax.broadcasted_iota(jnp.int32, x.shape, 1)
    et = jnp.sum(jnp.where(col == tgt, e, 0.0), axis=-1, keepdims=True)

    valid = tgt != _IGNORE
    logpt = jnp.log(et) - jnp.log(se)
    om = 1.0 - et / se
    focal = -(om * om) * logpt

    loss_ref[...] += jnp.sum(jnp.where(valid, focal, 0.0),
                             axis=0, keepdims=True)
    cnt_ref[...] += jnp.sum(jnp.where(valid, 1.0, 0.0),
                            axis=0, keepdims=True)

    @pl.when(i == n_steps - 1)
    def _store():
        lane = jax.lax.broadcasted_iota(jnp.int32, (1, 8, 128), 2)
        sub = jax.lax.broadcasted_iota(jnp.int32, (1, 8, 128), 1)
        row0 = sub == 0
        out_ref[...] = jnp.where(
            row0 & (lane == 0), loss_ref[...],
            jnp.where(row0 & (lane == 1), cnt_ref[...], 0.0))


@jax.jit
def kernel(logits, targets):
    N, C = logits.shape
    tgt2d = targets.astype(jnp.int32).reshape(N, 1)

    P = 2
    tile_n = 1024
    if N % (P * tile_n) != 0:
        if N % P == 0 and (N // P) % 8 == 0:
            tile_n = N // P
        else:
            P = 1
            tile_n = N
    steps = N // (P * tile_n)

    partials = pl.pallas_call(
        functools.partial(_focal_body, n_steps=steps),
        out_shape=jax.ShapeDtypeStruct((P, 8, 128), jnp.float32),
        grid=(P, steps),
        in_specs=[
            pl.BlockSpec((tile_n, C), lambda p, i: (p * steps + i, 0)),
            pl.BlockSpec((tile_n, 1), lambda p, i: (p * steps + i, 0)),
        ],
        out_specs=pl.BlockSpec((1, 8, 128), lambda p, i: (p, 0, 0)),
        scratch_shapes=[pltpu.VMEM((1, 1), jnp.float32),
                        pltpu.VMEM((1, 1), jnp.float32)],
        compiler_params=pltpu.CompilerParams(
            dimension_semantics=("parallel", "arbitrary"),
            vmem_limit_bytes=64 * 1024 * 1024),
    )(logits, tgt2d)

    loss_sum = jnp.sum(partials[:, 0, 0])
    valid_cnt = jnp.sum(partials[:, 0, 1])
    return loss_sum / valid_cnt


# speedup vs baseline: 1.2521x; 1.2521x over previous
import functools

import jax
import jax.numpy as jnp
from jax.experimental import pallas as pl
from jax.experimental.pallas import tpu as pltpu

_IGNORE = -100


def _focal_body(logits_ref, tgt_ref, out_ref, loss_ref, cnt_ref, *, n_steps):
    i = pl.program_id(1)

    @pl.when(i == 0)
    def _init():
        loss_ref[...] = jnp.zeros_like(loss_ref)
        cnt_ref[...] = jnp.zeros_like(cnt_ref)

    x = logits_ref[...]
    tgt = tgt_ref[...]

    m = jnp.max(x, axis=-1, keepdims=True)
    e = jnp.exp(x - m)
    se = jnp.sum(e, axis=-1, keepdims=True)
    col = jax.lax.broadcasted_iota(jnp.int32, x.shape, 1)
    et = jnp.sum(jnp.where(col == tgt, e, 0.0), axis=-1, keepdims=True)

    valid = tgt != _IGNORE
    logpt = jnp.log(et) - jnp.log(se)
    om = 1.0 - et / se
    focal = -(om * om) * logpt

    loss_ref[...] += jnp.sum(jnp.where(valid, focal, 0.0),
                             axis=0, keepdims=True)
    cnt_ref[...] += jnp.sum(jnp.where(valid, 1.0, 0.0),
                            axis=0, keepdims=True)

    @pl.when(i == n_steps - 1)
    def _store():
        lane = jax.lax.broadcasted_iota(jnp.int32, (1, 8, 128), 2)
        sub = jax.lax.broadcasted_iota(jnp.int32, (1, 8, 128), 1)
        row0 = sub == 0
        out_ref[...] = jnp.where(
            row0 & (lane == 0), loss_ref[...],
            jnp.where(row0 & (lane == 1), cnt_ref[...], 0.0))


@jax.jit
def kernel(logits, targets):
    N, C = logits.shape
    tgt2d = targets.astype(jnp.int32).reshape(N, 1)

    P = 2
    tile_n = 4096
    if N % (P * tile_n) != 0:
        if N % P == 0 and (N // P) % 8 == 0:
            tile_n = N // P
        else:
            P = 1
            tile_n = N
    steps = N // (P * tile_n)

    partials = pl.pallas_call(
        functools.partial(_focal_body, n_steps=steps),
        out_shape=jax.ShapeDtypeStruct((P, 8, 128), jnp.float32),
        grid=(P, steps),
        in_specs=[
            pl.BlockSpec((tile_n, C), lambda p, i: (p * steps + i, 0)),
            pl.BlockSpec((tile_n, 1), lambda p, i: (p * steps + i, 0)),
        ],
        out_specs=pl.BlockSpec((1, 8, 128), lambda p, i: (p, 0, 0)),
        scratch_shapes=[pltpu.VMEM((1, 1), jnp.float32),
                        pltpu.VMEM((1, 1), jnp.float32)],
        compiler_params=pltpu.CompilerParams(
            dimension_semantics=("parallel", "arbitrary"),
            vmem_limit_bytes=64 * 1024 * 1024),
    )(logits, tgt2d)

    loss_sum = jnp.sum(partials[:, 0, 0])
    valid_cnt = jnp.sum(partials[:, 0, 1])
    return loss_sum / valid_cnt


# speedup vs baseline: 1.5340x; 1.2252x over previous
import functools

import jax
import jax.numpy as jnp
from jax.experimental import pallas as pl
from jax.experimental.pallas import tpu as pltpu

_IGNORE = -100


def _focal_body(logits_ref, tgt_ref, out_ref, loss_ref, cnt_ref, *, n_steps):
    i = pl.program_id(1)

    @pl.when(i == 0)
    def _init():
        loss_ref[...] = jnp.zeros_like(loss_ref)
        cnt_ref[...] = jnp.zeros_like(cnt_ref)

    x = logits_ref[...]
    slab = tgt_ref[...]
    g, l = slab.shape
    tile_n = g * l

    srows = jnp.broadcast_to(slab[:, None, :], (g, l, l)).reshape(tile_n, l)
    lane = jax.lax.broadcasted_iota(jnp.int32, (tile_n, l), 1)
    rowm = jax.lax.broadcasted_iota(jnp.int32, (tile_n, l), 0) & (l - 1)
    tgt = jnp.sum(jnp.where(lane == rowm, srows, 0), axis=-1, keepdims=True)

    m = jnp.max(x, axis=-1, keepdims=True)
    e = jnp.exp(x - m)
    se = jnp.sum(e, axis=-1, keepdims=True)
    col = jax.lax.broadcasted_iota(jnp.int32, x.shape, 1)
    et = jnp.sum(jnp.where(col == tgt, e, 0.0), axis=-1, keepdims=True)

    valid = tgt != _IGNORE
    logpt = jnp.log(et) - jnp.log(se)
    om = 1.0 - et / se
    focal = -(om * om) * logpt

    loss_ref[...] += jnp.sum(jnp.where(valid, focal, 0.0),
                             axis=0, keepdims=True)
    cnt_ref[...] += jnp.sum(jnp.where(valid, 1.0, 0.0),
                            axis=0, keepdims=True)

    @pl.when(i == n_steps - 1)
    def _store():
        lane_o = jax.lax.broadcasted_iota(jnp.int32, (1, 8, 128), 2)
        sub_o = jax.lax.broadcasted_iota(jnp.int32, (1, 8, 128), 1)
        row0 = sub_o == 0
        out_ref[...] = jnp.where(
            row0 & (lane_o == 0), loss_ref[...],
            jnp.where(row0 & (lane_o == 1), cnt_ref[...], 0.0))


@jax.jit
def kernel(logits, targets):
    N, C = logits.shape
    tgtm = targets.astype(jnp.int32).reshape(N // 128, 128)

    P = 2
    tile_n = 4096
    if N % (P * tile_n) != 0:
        tile_n = N // P
    steps = N // (P * tile_n)
    rows128 = tile_n // 128

    partials = pl.pallas_call(
        functools.partial(_focal_body, n_steps=steps),
        out_shape=jax.ShapeDtypeStruct((P, 8, 128), jnp.float32),
        grid=(P, steps),
        in_specs=[
            pl.BlockSpec((tile_n, C), lambda p, i: (p * steps + i, 0)),
            pl.BlockSpec((rows128, 128), lambda p, i: (p * steps + i, 0)),
        ],
        out_specs=pl.BlockSpec((1, 8, 128), lambda p, i: (p, 0, 0)),
        scratch_shapes=[pltpu.VMEM((1, 1), jnp.float32),
                        pltpu.VMEM((1, 1), jnp.float32)],
        compiler_params=pltpu.CompilerParams(
            dimension_semantics=("parallel", "arbitrary"),
            vmem_limit_bytes=64 * 1024 * 1024),
    )(logits, tgtm)

    loss_sum = jnp.sum(partials[:, 0, 0])
    valid_cnt = jnp.sum(partials[:, 0, 1])
    return loss_sum / valid_cnt


# speedup vs baseline: 1.5546x; 1.0134x over previous
import functools

import jax
import jax.numpy as jnp
from jax.experimental import pallas as pl
from jax.experimental.pallas import tpu as pltpu

_IGNORE = -100


def _focal_body(logits_ref, tgt_ref, out_ref, loss_ref, cnt_ref, *, n_steps):
    i = pl.program_id(1)

    @pl.when(i == 0)
    def _init():
        loss_ref[...] = jnp.zeros_like(loss_ref)
        cnt_ref[...] = jnp.zeros_like(cnt_ref)

    x = logits_ref[...]
    slab = tgt_ref[...]
    g, l = slab.shape
    tile_n = g * l

    srows = jnp.broadcast_to(slab[:, None, :], (g, l, l)).reshape(tile_n, l)
    lane = jax.lax.broadcasted_iota(jnp.int32, (tile_n, l), 1)
    rowm = jax.lax.broadcasted_iota(jnp.int32, (tile_n, l), 0) & (l - 1)
    tgt = jnp.sum(jnp.where(lane == rowm, srows, 0), axis=-1, keepdims=True)

    m = jnp.max(x, axis=-1, keepdims=True)
    e = jnp.exp(x - m)
    se = jnp.sum(e, axis=-1, keepdims=True)
    col = jax.lax.broadcasted_iota(jnp.int32, x.shape, 1)
    et = jnp.sum(jnp.where(col == tgt, e, 0.0), axis=-1, keepdims=True)

    valid = tgt != _IGNORE
    logpt = jnp.log(et) - jnp.log(se)
    om = 1.0 - et / se
    focal = -(om * om) * logpt

    loss_ref[...] += jnp.sum(jnp.where(valid, focal, 0.0),
                             axis=0, keepdims=True)
    cnt_ref[...] += jnp.sum(jnp.where(valid, 1.0, 0.0),
                            axis=0, keepdims=True)

    @pl.when(i == n_steps - 1)
    def _store():
        lane_o = jax.lax.broadcasted_iota(jnp.int32, (1, 8, 128), 2)
        sub_o = jax.lax.broadcasted_iota(jnp.int32, (1, 8, 128), 1)
        row0 = sub_o == 0
        out_ref[...] = jnp.where(
            row0 & (lane_o == 0), loss_ref[...],
            jnp.where(row0 & (lane_o == 1), cnt_ref[...], 0.0))


@jax.jit
def kernel(logits, targets):
    N, C = logits.shape
    tgtm = targets.astype(jnp.int32).reshape(N // 128, 128)

    P = 2
    tile_n = 2048
    if N % (P * tile_n) != 0:
        tile_n = N // P
    steps = N // (P * tile_n)
    rows128 = tile_n // 128

    partials = pl.pallas_call(
        functools.partial(_focal_body, n_steps=steps),
        out_shape=jax.ShapeDtypeStruct((P, 8, 128), jnp.float32),
        grid=(P, steps),
        in_specs=[
            pl.BlockSpec((tile_n, C), lambda p, i: (p * steps + i, 0)),
            pl.BlockSpec((rows128, 128), lambda p, i: (p * steps + i, 0)),
        ],
        out_specs=pl.BlockSpec((1, 8, 128), lambda p, i: (p, 0, 0)),
        scratch_shapes=[pltpu.VMEM((1, 1), jnp.float32),
                        pltpu.VMEM((1, 1), jnp.float32)],
        compiler_params=pltpu.CompilerParams(
            dimension_semantics=("parallel", "arbitrary"),
            vmem_limit_bytes=64 * 1024 * 1024),
    )(logits, tgtm)

    loss_sum = jnp.sum(partials[:, 0, 0])
    valid_cnt = jnp.sum(partials[:, 0, 1])
    return loss_sum / valid_cnt
